# baseline (speedup 1.0000x reference)
import functools

import jax
import jax.numpy as jnp
from jax import lax
from jax.experimental import pallas as pl
from jax.experimental.pallas import tpu as pltpu
from jax.experimental.pallas import tpu_sc as plsc

EMBED_DIM = 128
NUM_ROWS = 256
BATCH = 16384
IDX_CHUNK = 128


@functools.cache
def _build_gather():
    info = plsc.get_sparse_core_info()
    num_cores, num_subcores, lanes = info.num_cores, info.num_subcores, info.num_lanes
    num_workers = num_cores * num_subcores
    b_per_w = BATCH // num_workers
    n_chunks = b_per_w // IDX_CHUNK
    mesh = plsc.VectorSubcoreMesh(core_axis_name="c", subcore_axis_name="s")

    @functools.partial(
        pl.kernel,
        mesh=mesh,
        out_type=jax.ShapeDtypeStruct((BATCH, EMBED_DIM), jnp.float32),
        scratch_types=[
            pltpu.VMEM((n_chunks, IDX_CHUNK), jnp.int32),
            pltpu.VMEM((b_per_w, EMBED_DIM), jnp.float32),
            pltpu.SemaphoreType.DMA((n_chunks,)),
            pltpu.SemaphoreType.DMA,
        ],
    )
    def gather_kernel(table_hbm, idx_hbm, out_hbm, idx_v, rows_v, gsem, wsem):
        wid = lax.axis_index("s") * num_cores + lax.axis_index("c")
        pltpu.sync_copy(idx_hbm.at[wid], idx_v)
        for j in range(n_chunks):
            for i in range(IDX_CHUNK // lanes):
                v = idx_v[j, pl.ds(i * lanes, lanes)]
                valid = (v >= 0) & (v < NUM_ROWS)
                idx_v[j, pl.ds(i * lanes, lanes)] = jnp.where(valid, v, NUM_ROWS)
        gathers = [
            pltpu.async_copy(
                table_hbm.at[idx_v.at[j]],
                rows_v.at[pl.ds(j * IDX_CHUNK, IDX_CHUNK)],
                gsem.at[j],
            )
            for j in range(n_chunks)
        ]
        writes = []
        for j in range(n_chunks):
            gathers[j].wait()
            writes.append(
                pltpu.async_copy(
                    rows_v.at[pl.ds(j * IDX_CHUNK, IDX_CHUNK)],
                    out_hbm.at[pl.ds(wid * b_per_w + j * IDX_CHUNK, IDX_CHUNK)],
                    wsem,
                )
            )
        for w in writes:
            w.wait()

    return gather_kernel, num_workers, n_chunks


def kernel(indices, tier0, tier1, tier2):
    gather, num_workers, n_chunks = _build_gather()
    table = jnp.concatenate(
        [tier0, tier1, tier2, jnp.zeros((1, EMBED_DIM), jnp.float32)], axis=0
    )
    idx = indices.astype(jnp.int32).reshape(num_workers, n_chunks, IDX_CHUNK)
    return gather(table, idx)


# speedup vs baseline: 1.0058x; 1.0058x over previous
import functools

import jax
import jax.numpy as jnp
from jax import lax
from jax.experimental import pallas as pl
from jax.experimental.pallas import tpu as pltpu
from jax.experimental.pallas import tpu_sc as plsc

EMBED_DIM = 128
NUM_ROWS = 256
BATCH = 16384
IDX_CHUNK = 128


@functools.cache
def _build_gather():
    info = plsc.get_sparse_core_info()
    num_cores, num_subcores, lanes = info.num_cores, info.num_subcores, info.num_lanes
    num_workers = num_cores * num_subcores
    b_per_w = BATCH // num_workers
    n_chunks = b_per_w // IDX_CHUNK
    mesh = plsc.VectorSubcoreMesh(core_axis_name="c", subcore_axis_name="s")

    @functools.partial(
        pl.kernel,
        mesh=mesh,
        out_type=jax.ShapeDtypeStruct((BATCH, EMBED_DIM), jnp.float32),
        scratch_types=[
            pltpu.VMEM((n_chunks, IDX_CHUNK), jnp.int32),
            pltpu.VMEM((b_per_w, EMBED_DIM), jnp.float32),
            pltpu.SemaphoreType.DMA((n_chunks,)),
            pltpu.SemaphoreType.DMA,
        ],
    )
    def gather_kernel(table_hbm, idx_hbm, out_hbm, idx_v, rows_v, gsem, wsem):
        wid = lax.axis_index("s") * num_cores + lax.axis_index("c")
        pltpu.sync_copy(idx_hbm.at[wid], idx_v)
        gathers = [
            pltpu.async_copy(
                table_hbm.at[idx_v.at[j]],
                rows_v.at[pl.ds(j * IDX_CHUNK, IDX_CHUNK)],
                gsem.at[j],
            )
            for j in range(n_chunks)
        ]
        writes = []
        for j in range(n_chunks):
            gathers[j].wait()
            writes.append(
                pltpu.async_copy(
                    rows_v.at[pl.ds(j * IDX_CHUNK, IDX_CHUNK)],
                    out_hbm.at[pl.ds(wid * b_per_w + j * IDX_CHUNK, IDX_CHUNK)],
                    wsem,
                )
            )
        for w in writes:
            w.wait()

    return gather_kernel, num_workers, n_chunks


def kernel(indices, tier0, tier1, tier2):
    gather, num_workers, n_chunks = _build_gather()
    table = jnp.concatenate(
        [tier0, tier1, tier2, jnp.zeros((1, EMBED_DIM), jnp.float32)], axis=0
    )
    idx = indices.astype(jnp.int32).reshape(num_workers, n_chunks, IDX_CHUNK)
    return gather(table, idx)


# speedup vs baseline: 1.3367x; 1.3290x over previous
import functools

import jax
import jax.numpy as jnp
from jax import lax
from jax.experimental import pallas as pl
from jax.experimental.pallas import tpu as pltpu
from jax.experimental.pallas import tpu_sc as plsc

EMBED_DIM = 128
NUM_ROWS = 256
BATCH = 16384
IDX_CHUNK = 128


@functools.cache
def _build_gather():
    info = plsc.get_sparse_core_info()
    num_cores, num_subcores, lanes = info.num_cores, info.num_subcores, info.num_lanes
    num_workers = num_cores * num_subcores
    b_per_w = BATCH // num_workers
    n_chunks = b_per_w // IDX_CHUNK
    mesh = plsc.VectorSubcoreMesh(core_axis_name="c", subcore_axis_name="s")

    @functools.partial(
        pl.kernel,
        mesh=mesh,
        out_type=jax.ShapeDtypeStruct((BATCH, EMBED_DIM), jnp.float32),
        scratch_types=[
            pltpu.VMEM((n_chunks, IDX_CHUNK), jnp.int32),
            pltpu.VMEM((b_per_w, EMBED_DIM), jnp.float32),
            pltpu.VMEM_SHARED((NUM_ROWS + 1, EMBED_DIM), jnp.float32),
            pltpu.SemaphoreType.DMA((n_chunks,)),
            pltpu.SemaphoreType.DMA,
        ],
    )
    def gather_kernel(table_hbm, idx_hbm, out_hbm, idx_v, rows_v, table_sh, gsem, wsem):
        wid = lax.axis_index("s") * num_cores + lax.axis_index("c")
        @pl.when(lax.axis_index("s") == 0)
        def _load_table():
            pltpu.sync_copy(table_hbm, table_sh)

        pltpu.sync_copy(idx_hbm.at[wid], idx_v)
        plsc.subcore_barrier()
        gathers = [
            pltpu.async_copy(
                table_sh.at[idx_v.at[j]],
                rows_v.at[pl.ds(j * IDX_CHUNK, IDX_CHUNK)],
                gsem.at[j],
            )
            for j in range(n_chunks)
        ]
        writes = []
        for j in range(n_chunks):
            gathers[j].wait()
            writes.append(
                pltpu.async_copy(
                    rows_v.at[pl.ds(j * IDX_CHUNK, IDX_CHUNK)],
                    out_hbm.at[pl.ds(wid * b_per_w + j * IDX_CHUNK, IDX_CHUNK)],
                    wsem,
                )
            )
        for w in writes:
            w.wait()

    return gather_kernel, num_workers, n_chunks


def kernel(indices, tier0, tier1, tier2):
    gather, num_workers, n_chunks = _build_gather()
    table = jnp.concatenate(
        [tier0, tier1, tier2, jnp.zeros((1, EMBED_DIM), jnp.float32)], axis=0
    )
    idx = indices.astype(jnp.int32).reshape(num_workers, n_chunks, IDX_CHUNK)
    return gather(table, idx)
